# Initial kernel scaffold; baseline (speedup 1.0000x reference)
import jax
import jax.numpy as jnp
from jax.experimental import pallas as pl
from jax.experimental.pallas import tpu as pltpu


def _copy_body(perm_ref, x_ref, o_ref):
    o_ref[...] = x_ref[...]


def kernel(x, ldj, permutation):
    B, C, H, W = x.shape
    HW = H * W
    x2 = x.reshape(B, C, HW)
    out = pl.pallas_call(
        _copy_body,
        grid_spec=pltpu.PrefetchScalarGridSpec(
            num_scalar_prefetch=1,
            grid=(C,),
            in_specs=[
                pl.BlockSpec((B, 1, HW), lambda c, perm: (0, perm[c], 0)),
            ],
            out_specs=pl.BlockSpec((B, 1, HW), lambda c, perm: (0, c, 0)),
        ),
        out_shape=jax.ShapeDtypeStruct((B, C, HW), x.dtype),
    )(permutation, x2)
    return out.reshape(B, C, H, W), ldj


# speedup vs baseline: 2.1123x; 2.1123x over previous
import jax
import jax.numpy as jnp
from jax.experimental import pallas as pl
from jax.experimental.pallas import tpu as pltpu


def _copy_body(perm_ref, x_ref, o_ref):
    o_ref[...] = x_ref[...]


def kernel(x, ldj, permutation):
    B, C, H, W = x.shape
    out = pl.pallas_call(
        _copy_body,
        grid_spec=pltpu.PrefetchScalarGridSpec(
            num_scalar_prefetch=1,
            grid=(C,),
            in_specs=[
                pl.BlockSpec((B, 1, H, W), lambda c, perm: (0, perm[c], 0, 0)),
            ],
            out_specs=pl.BlockSpec((B, 1, H, W), lambda c, perm: (0, c, 0, 0)),
        ),
        out_shape=jax.ShapeDtypeStruct((B, C, H, W), x.dtype),
    )(permutation, x)
    return out, ldj


# speedup vs baseline: 2.1265x; 1.0068x over previous
import jax
import jax.numpy as jnp
from jax.experimental import pallas as pl
from jax.experimental.pallas import tpu as pltpu


def _copy_body(perm_ref, x_ref, o_ref):
    o_ref[...] = x_ref[...]


def kernel(x, ldj, permutation):
    B, C, H, W = x.shape
    out = pl.pallas_call(
        _copy_body,
        grid_spec=pltpu.PrefetchScalarGridSpec(
            num_scalar_prefetch=1,
            grid=(C,),
            in_specs=[
                pl.BlockSpec((B, 1, H, W), lambda c, perm: (0, perm[c], 0, 0)),
            ],
            out_specs=pl.BlockSpec((B, 1, H, W), lambda c, perm: (0, c, 0, 0)),
        ),
        out_shape=jax.ShapeDtypeStruct((B, C, H, W), x.dtype),
        compiler_params=pltpu.CompilerParams(
            dimension_semantics=("parallel",),
        ),
    )(permutation, x)
    return out, ldj
